# baseline (speedup 1.0000x reference)
import jax
import jax.numpy as jnp
from jax import lax
from jax.experimental import pallas as pl
from jax.experimental.pallas import tpu as pltpu
from jax.experimental.pallas import tpu_sc as plsc

ZCH_N = 1000000
DIM = 64
N_BUCKETS = 4
BUCKET_SZ = ZCH_N // N_BUCKETS
NUM_N = 16384

NC = 2
NS = 16
L = 16

IDS_PER_TILE = NUM_N // NS
CHUNK = 128
NCHUNK = IDS_PER_TILE // CHUNK
G_PER_TILE = NUM_N // (NC * NS)
GCHUNK = G_PER_TILE // CHUNK
META_CHUNK = 62496
META_LAST = ZCH_N - 15 * META_CHUNK
NBOUNCE = 4
SUB = META_CHUNK // NBOUNCE
SUB_LAST = META_LAST // NBOUNCE

_MESH = dict(core_axis_name="c", subcore_axis_name="s",
             num_cores=NC, num_subcores=NS)


def _remap16(v):
    h = v.astype(jnp.uint32) * jnp.uint32(2654435761)
    bucket = h & jnp.uint32(N_BUCKETS - 1)
    offset = (h >> jnp.uint32(2)) % jnp.uint32(BUCKET_SZ)
    return (bucket * jnp.uint32(BUCKET_SZ) + offset).astype(jnp.int32)


def _meta_body(values_hbm, meta_hbm, remap_hbm, meta_out_hbm,
               vals_v, idx_f, idx_v, ones_v, bnc_v, meta_sh):
    cid = lax.axis_index("c")
    sid = lax.axis_index("s")
    base = sid * IDS_PER_TILE

    def _stage_in(off, sub):
        for j in range(NBOUNCE):
            o = off + j * sub
            pltpu.sync_copy(meta_hbm.at[pl.ds(o, sub)], bnc_v.at[pl.ds(0, sub)])
            pltpu.sync_copy(bnc_v.at[pl.ds(0, sub)], meta_sh.at[pl.ds(o, sub)])

    def _stage_out(off, sub):
        for j in range(NBOUNCE):
            o = off + j * sub
            pltpu.sync_copy(meta_sh.at[pl.ds(o, sub)], bnc_v.at[pl.ds(0, sub)])
            pltpu.sync_copy(bnc_v.at[pl.ds(0, sub)], meta_out_hbm.at[pl.ds(o, sub)])

    @pl.when(cid == 0)
    def _phase_a():
        pltpu.sync_copy(values_hbm.at[pl.ds(base, IDS_PER_TILE)], vals_v)
        for j in range(NCHUNK):
            for i in range(CHUNK // L):
                s = j * CHUNK + i * L
                r = _remap16(vals_v[pl.ds(s, L)])
                idx_f[pl.ds(s, L)] = r
                idx_v[j, pl.ds(i * L, L)] = r
                ones_v[j, pl.ds(i * L, L)] = jnp.full((L,), 1.0, jnp.float32)
        pltpu.sync_copy(idx_f, remap_hbm.at[pl.ds(base, IDS_PER_TILE)])

        @pl.when(sid < NS - 1)
        def _():
            _stage_in(sid * META_CHUNK, SUB)

        @pl.when(sid == NS - 1)
        def _():
            _stage_in((NS - 1) * META_CHUNK, SUB_LAST)

    plsc.subcore_barrier()

    @pl.when(cid == 0)
    def _phase_scatter():
        for j in range(NCHUNK):
            pltpu.sync_copy(ones_v.at[j], meta_sh.at[idx_v.at[j]], add=True)

    plsc.subcore_barrier()

    @pl.when(cid == 0)
    def _phase_writeback():
        @pl.when(sid < NS - 1)
        def _():
            _stage_out(sid * META_CHUNK, SUB)

        @pl.when(sid == NS - 1)
        def _():
            _stage_out((NS - 1) * META_CHUNK, SUB_LAST)


def _gather_body(values_hbm, table_hbm, emb_hbm,
                 vals_v, idx_v, rows_v, gsem):
    cid = lax.axis_index("c")
    sid = lax.axis_index("s")
    wid = sid * NC + cid
    base = wid * G_PER_TILE

    pltpu.sync_copy(values_hbm.at[pl.ds(base, G_PER_TILE)], vals_v)
    for j in range(GCHUNK):
        for i in range(CHUNK // L):
            s = j * CHUNK + i * L
            idx_v[j, pl.ds(i * L, L)] = _remap16(vals_v[pl.ds(s, L)])

    copies = []
    for j in range(GCHUNK):
        copies.append(pltpu.async_copy(
            table_hbm.at[idx_v.at[j]],
            rows_v.at[pl.ds(j * CHUNK, CHUNK)], gsem))
    for c in copies:
        c.wait()
    pltpu.sync_copy(rows_v, emb_hbm.at[pl.ds(base, G_PER_TILE)])


def kernel(values, lengths, table, meta):
    del lengths
    meta_k = pl.kernel(
        _meta_body,
        out_type=(
            jax.ShapeDtypeStruct((NUM_N,), jnp.int32),
            jax.ShapeDtypeStruct((ZCH_N,), jnp.float32),
        ),
        mesh=plsc.VectorSubcoreMesh(**_MESH),
        scratch_types=[
            pltpu.VMEM((IDS_PER_TILE,), jnp.int32),
            pltpu.VMEM((IDS_PER_TILE,), jnp.int32),
            pltpu.VMEM((NCHUNK, CHUNK), jnp.int32),
            pltpu.VMEM((NCHUNK, CHUNK), jnp.float32),
            pltpu.VMEM((SUB_LAST,), jnp.float32),
            pltpu.VMEM_SHARED((ZCH_N,), jnp.float32),
        ],
    )
    gather_k = pl.kernel(
        _gather_body,
        out_type=jax.ShapeDtypeStruct((NUM_N, DIM), jnp.float32),
        mesh=plsc.VectorSubcoreMesh(**_MESH),
        compiler_params=pltpu.CompilerParams(use_tc_tiling_on_sc=False),
        scratch_types=[
            pltpu.VMEM((G_PER_TILE,), jnp.int32),
            pltpu.VMEM((GCHUNK, CHUNK), jnp.int32),
            pltpu.VMEM((G_PER_TILE, DIM), jnp.float32),
            pltpu.SemaphoreType.DMA,
        ],
    )
    remapped, meta_new = meta_k(values, meta)
    emb = gather_k(values, table)
    return emb, remapped, meta_new


# speedup vs baseline: 1.4852x; 1.4852x over previous
import jax
import jax.numpy as jnp
from jax import lax
from jax.experimental import pallas as pl
from jax.experimental.pallas import tpu as pltpu
from jax.experimental.pallas import tpu_sc as plsc

ZCH_N = 1000000
DIM = 64
N_BUCKETS = 4
BUCKET_SZ = ZCH_N // N_BUCKETS
NUM_N = 16384

NC = 2
NS = 16
L = 16

IDS_PER_TILE = NUM_N // NS
CHUNK = 128
NCHUNK = IDS_PER_TILE // CHUNK
G_PER_TILE = NUM_N // (NC * NS)
GCHUNK = G_PER_TILE // CHUNK
META_CHUNK = 62496
META_LAST = ZCH_N - 15 * META_CHUNK
NBOUNCE = 4
SUB = META_CHUNK // NBOUNCE
SUB_LAST = META_LAST // NBOUNCE

_MESH = dict(core_axis_name="c", subcore_axis_name="s",
             num_cores=NC, num_subcores=NS)


def _remap16(v):
    h = v.astype(jnp.uint32) * jnp.uint32(2654435761)
    bucket = h & jnp.uint32(N_BUCKETS - 1)
    offset = (h >> jnp.uint32(2)) % jnp.uint32(BUCKET_SZ)
    return (bucket * jnp.uint32(BUCKET_SZ) + offset).astype(jnp.int32)


def _meta_body(values_hbm, meta_hbm, remap_hbm, meta_out_hbm,
               vals_v, idx_f, idx_v, ones_v, bnc_v, meta_sh):
    cid = lax.axis_index("c")
    sid = lax.axis_index("s")
    base = sid * IDS_PER_TILE

    def _stage_in(off, sub):
        for j in range(NBOUNCE):
            o = off + j * sub
            pltpu.sync_copy(meta_hbm.at[pl.ds(o, sub)], bnc_v.at[pl.ds(0, sub)])
            pltpu.sync_copy(bnc_v.at[pl.ds(0, sub)], meta_sh.at[pl.ds(o, sub)])

    def _stage_out(off, sub):
        for j in range(NBOUNCE):
            o = off + j * sub
            pltpu.sync_copy(meta_sh.at[pl.ds(o, sub)], bnc_v.at[pl.ds(0, sub)])
            pltpu.sync_copy(bnc_v.at[pl.ds(0, sub)], meta_out_hbm.at[pl.ds(o, sub)])

    @pl.when(cid == 0)
    def _phase_a():
        pltpu.sync_copy(values_hbm.at[pl.ds(base, IDS_PER_TILE)], vals_v)
        for j in range(NCHUNK):
            for i in range(CHUNK // L):
                s = j * CHUNK + i * L
                r = _remap16(vals_v[pl.ds(s, L)])
                idx_f[pl.ds(s, L)] = r
                idx_v[j, pl.ds(i * L, L)] = r
                ones_v[j, pl.ds(i * L, L)] = jnp.full((L,), 1.0, jnp.float32)
        pltpu.sync_copy(idx_f, remap_hbm.at[pl.ds(base, IDS_PER_TILE)])

        @pl.when(sid < NS - 1)
        def _():
            _stage_in(sid * META_CHUNK, SUB)

        @pl.when(sid == NS - 1)
        def _():
            _stage_in((NS - 1) * META_CHUNK, SUB_LAST)

    plsc.subcore_barrier()

    @pl.when(cid == 0)
    def _phase_scatter():
        for j in range(NCHUNK):
            pltpu.sync_copy(ones_v.at[j], meta_sh.at[idx_v.at[j]], add=True)

    plsc.subcore_barrier()

    @pl.when(cid == 0)
    def _phase_writeback():
        @pl.when(sid < NS - 1)
        def _():
            _stage_out(sid * META_CHUNK, SUB)

        @pl.when(sid == NS - 1)
        def _():
            _stage_out((NS - 1) * META_CHUNK, SUB_LAST)


GB = 16


def _gather_body(values_hbm, table_hbm, emb_hbm,
                 vals_v, idx_f, blk_v, rows_v, gsem):
    cid = lax.axis_index("c")
    sid = lax.axis_index("s")
    wid = sid * NC + cid
    base = wid * G_PER_TILE

    pltpu.sync_copy(values_hbm.at[pl.ds(base, G_PER_TILE)], vals_v)
    for i in range(G_PER_TILE // L):
        idx_f[pl.ds(i * L, L)] = _remap16(vals_v[pl.ds(i * L, L)])
    obase = base * DIM

    def _gather_chunk(c, carry):
        cb = c * GB
        rv = idx_f[pl.ds(cb, L)]
        copies = []
        ks = []
        for i in range(GB):
            r = rv[i]
            ks.append(r & 7)
            rblk = pl.multiple_of(r - ks[i], 8)
            copies.append(pltpu.async_copy(
                table_hbm.at[pl.ds(rblk, 8), :], blk_v.at[i], gsem))
        for cp in copies:
            cp.wait()
        for i in range(GB):
            for j in range(DIM // L):
                rows_v[pl.ds(i * DIM + j * L, L)] = blk_v[i, ks[i], pl.ds(j * L, L)]
        pltpu.sync_copy(
            rows_v, emb_hbm.at[pl.ds(obase + cb * DIM, GB * DIM)])
        return carry

    lax.fori_loop(0, G_PER_TILE // GB, _gather_chunk, 0)


def kernel(values, lengths, table, meta):
    del lengths
    meta_k = pl.kernel(
        _meta_body,
        out_type=(
            jax.ShapeDtypeStruct((NUM_N,), jnp.int32),
            jax.ShapeDtypeStruct((ZCH_N,), jnp.float32),
        ),
        mesh=plsc.VectorSubcoreMesh(**_MESH),
        scratch_types=[
            pltpu.VMEM((IDS_PER_TILE,), jnp.int32),
            pltpu.VMEM((IDS_PER_TILE,), jnp.int32),
            pltpu.VMEM((NCHUNK, CHUNK), jnp.int32),
            pltpu.VMEM((NCHUNK, CHUNK), jnp.float32),
            pltpu.VMEM((SUB_LAST,), jnp.float32),
            pltpu.VMEM_SHARED((ZCH_N,), jnp.float32),
        ],
    )
    gather_k = pl.kernel(
        _gather_body,
        out_type=jax.ShapeDtypeStruct((NUM_N * DIM,), jnp.float32),
        mesh=plsc.VectorSubcoreMesh(**_MESH),
        scratch_types=[
            pltpu.VMEM((G_PER_TILE,), jnp.int32),
            pltpu.VMEM((G_PER_TILE,), jnp.int32),
            pltpu.VMEM((GB, 8, DIM), jnp.float32),
            pltpu.VMEM((GB * DIM,), jnp.float32),
            pltpu.SemaphoreType.DMA,
        ],
    )
    remapped, meta_new = meta_k(values, meta)
    emb_flat = gather_k(values, table)
    return emb_flat.reshape(NUM_N, DIM), remapped, meta_new


# speedup vs baseline: 1.5628x; 1.0522x over previous
import jax
import jax.numpy as jnp
from jax import lax
from jax.experimental import pallas as pl
from jax.experimental.pallas import tpu as pltpu
from jax.experimental.pallas import tpu_sc as plsc

ZCH_N = 1000000
DIM = 64
N_BUCKETS = 4
BUCKET_SZ = ZCH_N // N_BUCKETS
NUM_N = 16384

NC = 2
NS = 16
L = 16

IDS_PER_TILE = NUM_N // NS
CHUNK = 128
NCHUNK = IDS_PER_TILE // CHUNK
G_PER_TILE = NUM_N // (NC * NS)
GCHUNK = G_PER_TILE // CHUNK
META_CHUNK = 62496
META_LAST = ZCH_N - 15 * META_CHUNK
NBOUNCE = 4
SUB = META_CHUNK // NBOUNCE
SUB_LAST = META_LAST // NBOUNCE

_MESH = dict(core_axis_name="c", subcore_axis_name="s",
             num_cores=NC, num_subcores=NS)


def _remap16(v):
    h = v.astype(jnp.uint32) * jnp.uint32(2654435761)
    bucket = h & jnp.uint32(N_BUCKETS - 1)
    offset = (h >> jnp.uint32(2)) % jnp.uint32(BUCKET_SZ)
    return (bucket * jnp.uint32(BUCKET_SZ) + offset).astype(jnp.int32)


def _meta_body(values_hbm, meta_hbm, remap_hbm, meta_out_hbm,
               vals_v, idx_f, idx_v, ones_v, bnc_v, meta_sh):
    cid = lax.axis_index("c")
    sid = lax.axis_index("s")
    base = sid * IDS_PER_TILE

    def _stage_in(off, sub):
        for j in range(NBOUNCE):
            o = off + j * sub
            pltpu.sync_copy(meta_hbm.at[pl.ds(o, sub)], bnc_v.at[pl.ds(0, sub)])
            pltpu.sync_copy(bnc_v.at[pl.ds(0, sub)], meta_sh.at[pl.ds(o, sub)])

    def _stage_out(off, sub):
        for j in range(NBOUNCE):
            o = off + j * sub
            pltpu.sync_copy(meta_sh.at[pl.ds(o, sub)], bnc_v.at[pl.ds(0, sub)])
            pltpu.sync_copy(bnc_v.at[pl.ds(0, sub)], meta_out_hbm.at[pl.ds(o, sub)])

    @pl.when(cid == 0)
    def _phase_a():
        pltpu.sync_copy(values_hbm.at[pl.ds(base, IDS_PER_TILE)], vals_v)
        for j in range(NCHUNK):
            for i in range(CHUNK // L):
                s = j * CHUNK + i * L
                r = _remap16(vals_v[pl.ds(s, L)])
                idx_f[pl.ds(s, L)] = r
                idx_v[j, pl.ds(i * L, L)] = r
                ones_v[j, pl.ds(i * L, L)] = jnp.full((L,), 1.0, jnp.float32)
        pltpu.sync_copy(idx_f, remap_hbm.at[pl.ds(base, IDS_PER_TILE)])

        @pl.when(sid < NS - 1)
        def _():
            _stage_in(sid * META_CHUNK, SUB)

        @pl.when(sid == NS - 1)
        def _():
            _stage_in((NS - 1) * META_CHUNK, SUB_LAST)

    plsc.subcore_barrier()

    @pl.when(cid == 0)
    def _phase_scatter():
        for j in range(NCHUNK):
            pltpu.sync_copy(ones_v.at[j], meta_sh.at[idx_v.at[j]], add=True)

    plsc.subcore_barrier()

    @pl.when(cid == 0)
    def _phase_writeback():
        @pl.when(sid < NS - 1)
        def _():
            _stage_out(sid * META_CHUNK, SUB)

        @pl.when(sid == NS - 1)
        def _():
            _stage_out((NS - 1) * META_CHUNK, SUB_LAST)


GB = 16


def _gather_body(values_hbm, table_hbm, emb_hbm,
                 vals_v, idx_f, blk_v, blk2_v, rows_v, gsem, gsem2):
    cid = lax.axis_index("c")
    sid = lax.axis_index("s")
    wid = sid * NC + cid
    base = wid * G_PER_TILE

    pltpu.sync_copy(values_hbm.at[pl.ds(base, G_PER_TILE)], vals_v)
    for i in range(G_PER_TILE // L):
        idx_f[pl.ds(i * L, L)] = _remap16(vals_v[pl.ds(i * L, L)])
    obase = base * DIM

    NCHK = G_PER_TILE // GB

    def _fire(c, buf, sem):
        rv = idx_f[pl.ds(c * GB, L)]
        for i in range(GB):
            r = rv[i]
            rblk = pl.multiple_of(r - (r & 7), 8)
            pltpu.async_copy(table_hbm.at[pl.ds(rblk, 8), :], buf.at[i], sem)

    def _drain_extract(c, buf, sem):
        rv = idx_f[pl.ds(c * GB, L)]
        ks = []
        for i in range(GB):
            r = rv[i]
            ks.append(r & 7)
            rblk = pl.multiple_of(r - ks[i], 8)
            pltpu.make_async_copy(
                table_hbm.at[pl.ds(rblk, 8), :], buf.at[i], sem).wait()
        for i in range(GB):
            for j in range(DIM // L):
                rows_v[pl.ds(i * DIM + j * L, L)] = buf[i, ks[i], pl.ds(j * L, L)]
        pltpu.sync_copy(
            rows_v, emb_hbm.at[pl.ds(obase + c * GB * DIM, GB * DIM)])

    _fire(0, blk_v, gsem)

    def _pair(k, carry):
        c = 2 * k
        _fire(c + 1, blk2_v, gsem2)
        _drain_extract(c, blk_v, gsem)

        @pl.when(k < NCHK // 2 - 1)
        def _():
            _fire(c + 2, blk_v, gsem)

        _drain_extract(c + 1, blk2_v, gsem2)
        return carry

    lax.fori_loop(0, NCHK // 2, _pair, 0)


def kernel(values, lengths, table, meta):
    del lengths
    meta_k = pl.kernel(
        _meta_body,
        out_type=(
            jax.ShapeDtypeStruct((NUM_N,), jnp.int32),
            jax.ShapeDtypeStruct((ZCH_N,), jnp.float32),
        ),
        mesh=plsc.VectorSubcoreMesh(**_MESH),
        scratch_types=[
            pltpu.VMEM((IDS_PER_TILE,), jnp.int32),
            pltpu.VMEM((IDS_PER_TILE,), jnp.int32),
            pltpu.VMEM((NCHUNK, CHUNK), jnp.int32),
            pltpu.VMEM((NCHUNK, CHUNK), jnp.float32),
            pltpu.VMEM((SUB_LAST,), jnp.float32),
            pltpu.VMEM_SHARED((ZCH_N,), jnp.float32),
        ],
    )
    gather_k = pl.kernel(
        _gather_body,
        out_type=jax.ShapeDtypeStruct((NUM_N * DIM,), jnp.float32),
        mesh=plsc.VectorSubcoreMesh(**_MESH),
        scratch_types=[
            pltpu.VMEM((G_PER_TILE,), jnp.int32),
            pltpu.VMEM((G_PER_TILE,), jnp.int32),
            pltpu.VMEM((GB, 8, DIM), jnp.float32),
            pltpu.VMEM((GB, 8, DIM), jnp.float32),
            pltpu.VMEM((GB * DIM,), jnp.float32),
            pltpu.SemaphoreType.DMA,
            pltpu.SemaphoreType.DMA,
        ],
    )
    remapped, meta_new = meta_k(values, meta)
    emb_flat = gather_k(values, table)
    return emb_flat.reshape(NUM_N, DIM), remapped, meta_new


# speedup vs baseline: 1.5883x; 1.0163x over previous
import jax
import jax.numpy as jnp
from jax import lax
from jax.experimental import pallas as pl
from jax.experimental.pallas import tpu as pltpu
from jax.experimental.pallas import tpu_sc as plsc

ZCH_N = 1000000
DIM = 64
N_BUCKETS = 4
BUCKET_SZ = ZCH_N // N_BUCKETS
NUM_N = 16384

NC = 2
NS = 16
L = 16

IDS_PER_TILE = NUM_N // NS
CHUNK = 128
NCHUNK = IDS_PER_TILE // CHUNK
G_PER_TILE = NUM_N // (NC * NS)
GCHUNK = G_PER_TILE // CHUNK
META_CHUNK = 62496
META_LAST = ZCH_N - 15 * META_CHUNK
NBOUNCE = 4
SUB = META_CHUNK // NBOUNCE
SUB_LAST = META_LAST // NBOUNCE

_MESH = dict(core_axis_name="c", subcore_axis_name="s",
             num_cores=NC, num_subcores=NS)


def _remap16(v):
    h = v.astype(jnp.uint32) * jnp.uint32(2654435761)
    bucket = h & jnp.uint32(N_BUCKETS - 1)
    offset = (h >> jnp.uint32(2)) % jnp.uint32(BUCKET_SZ)
    return (bucket * jnp.uint32(BUCKET_SZ) + offset).astype(jnp.int32)


def _meta_body(values_hbm, meta_hbm, remap_hbm, meta_out_hbm,
               vals_v, idx_f, idx_v, ones_v, bnc_v, meta_sh):
    cid = lax.axis_index("c")
    sid = lax.axis_index("s")
    base = sid * IDS_PER_TILE

    def _stage_in(off, sub):
        for j in range(NBOUNCE):
            o = off + j * sub
            pltpu.sync_copy(meta_hbm.at[pl.ds(o, sub)], bnc_v.at[pl.ds(0, sub)])
            pltpu.sync_copy(bnc_v.at[pl.ds(0, sub)], meta_sh.at[pl.ds(o, sub)])

    def _stage_out(off, sub):
        for j in range(NBOUNCE):
            o = off + j * sub
            pltpu.sync_copy(meta_sh.at[pl.ds(o, sub)], bnc_v.at[pl.ds(0, sub)])
            pltpu.sync_copy(bnc_v.at[pl.ds(0, sub)], meta_out_hbm.at[pl.ds(o, sub)])

    @pl.when(cid == 0)
    def _phase_a():
        pltpu.sync_copy(values_hbm.at[pl.ds(base, IDS_PER_TILE)], vals_v)
        for j in range(NCHUNK):
            for i in range(CHUNK // L):
                s = j * CHUNK + i * L
                r = _remap16(vals_v[pl.ds(s, L)])
                idx_f[pl.ds(s, L)] = r
                idx_v[j, pl.ds(i * L, L)] = r
                ones_v[j, pl.ds(i * L, L)] = jnp.full((L,), 1.0, jnp.float32)
        pltpu.sync_copy(idx_f, remap_hbm.at[pl.ds(base, IDS_PER_TILE)])

        @pl.when(sid < NS - 1)
        def _():
            _stage_in(sid * META_CHUNK, SUB)

        @pl.when(sid == NS - 1)
        def _():
            _stage_in((NS - 1) * META_CHUNK, SUB_LAST)

    plsc.subcore_barrier()

    @pl.when(cid == 0)
    def _phase_scatter():
        for j in range(NCHUNK):
            pltpu.sync_copy(ones_v.at[j], meta_sh.at[idx_v.at[j]], add=True)

    plsc.subcore_barrier()

    @pl.when(cid == 0)
    def _phase_writeback():
        @pl.when(sid < NS - 1)
        def _():
            _stage_out(sid * META_CHUNK, SUB)

        @pl.when(sid == NS - 1)
        def _():
            _stage_out((NS - 1) * META_CHUNK, SUB_LAST)


GB = 32


def _gather_body(values_hbm, table_hbm, emb_hbm,
                 vals_v, idx_f, blk_v, blk2_v, rows_v, gsem, gsem2):
    cid = lax.axis_index("c")
    sid = lax.axis_index("s")
    wid = sid * NC + cid
    base = wid * G_PER_TILE

    pltpu.sync_copy(values_hbm.at[pl.ds(base, G_PER_TILE)], vals_v)
    for i in range(G_PER_TILE // L):
        idx_f[pl.ds(i * L, L)] = _remap16(vals_v[pl.ds(i * L, L)])
    obase = base * DIM

    NCHK = G_PER_TILE // GB

    def _fire(c, buf, sem):
        for g in range(GB // L):
            rv = idx_f[pl.ds(c * GB + g * L, L)]
            for i in range(L):
                r = rv[i]
                rblk = pl.multiple_of(r - (r & 7), 8)
                pltpu.async_copy(table_hbm.at[pl.ds(rblk, 8), :],
                                 buf.at[g * L + i], sem)

    def _drain_extract(c, buf, sem):
        for g in range(GB // L):
            rv = idx_f[pl.ds(c * GB + g * L, L)]
            for i in range(L):
                r = rv[i]
                k = r & 7
                rblk = pl.multiple_of(r - k, 8)
                pltpu.make_async_copy(
                    table_hbm.at[pl.ds(rblk, 8), :],
                    buf.at[g * L + i], sem).wait()
                for j in range(DIM // L):
                    rows_v[pl.ds((g * L + i) * DIM + j * L, L)] =                         buf[g * L + i, k, pl.ds(j * L, L)]
        pltpu.sync_copy(
            rows_v, emb_hbm.at[pl.ds(obase + c * GB * DIM, GB * DIM)])

    _fire(0, blk_v, gsem)

    def _pair(k, carry):
        c = 2 * k
        _fire(c + 1, blk2_v, gsem2)
        _drain_extract(c, blk_v, gsem)

        @pl.when(k < NCHK // 2 - 1)
        def _():
            _fire(c + 2, blk_v, gsem)

        _drain_extract(c + 1, blk2_v, gsem2)
        return carry

    lax.fori_loop(0, NCHK // 2, _pair, 0)


def kernel(values, lengths, table, meta):
    del lengths
    meta_k = pl.kernel(
        _meta_body,
        out_type=(
            jax.ShapeDtypeStruct((NUM_N,), jnp.int32),
            jax.ShapeDtypeStruct((ZCH_N,), jnp.float32),
        ),
        mesh=plsc.VectorSubcoreMesh(**_MESH),
        scratch_types=[
            pltpu.VMEM((IDS_PER_TILE,), jnp.int32),
            pltpu.VMEM((IDS_PER_TILE,), jnp.int32),
            pltpu.VMEM((NCHUNK, CHUNK), jnp.int32),
            pltpu.VMEM((NCHUNK, CHUNK), jnp.float32),
            pltpu.VMEM((SUB_LAST,), jnp.float32),
            pltpu.VMEM_SHARED((ZCH_N,), jnp.float32),
        ],
    )
    gather_k = pl.kernel(
        _gather_body,
        out_type=jax.ShapeDtypeStruct((NUM_N * DIM,), jnp.float32),
        mesh=plsc.VectorSubcoreMesh(**_MESH),
        scratch_types=[
            pltpu.VMEM((G_PER_TILE,), jnp.int32),
            pltpu.VMEM((G_PER_TILE,), jnp.int32),
            pltpu.VMEM((GB, 8, DIM), jnp.float32),
            pltpu.VMEM((GB, 8, DIM), jnp.float32),
            pltpu.VMEM((GB * DIM,), jnp.float32),
            pltpu.SemaphoreType.DMA,
            pltpu.SemaphoreType.DMA,
        ],
    )
    emb_flat = gather_k(values, table)
    remapped, meta_new = meta_k(values, meta)
    return emb_flat.reshape(NUM_N, DIM), remapped, meta_new
